# baseline (speedup 1.0000x reference)
import jax
import jax.numpy as jnp
from jax.experimental import pallas as pl

_VOCAB = 1_000_000
_EMB = 64
_BLOCK_ROWS = 8_000


def _copy_body(in_ref, out_ref):
    out_ref[...] = in_ref[...]


def kernel(lang, W_emb):
    del lang
    out = pl.pallas_call(
        _copy_body,
        grid=(_VOCAB // _BLOCK_ROWS,),
        in_specs=[pl.BlockSpec((_BLOCK_ROWS, _EMB), lambda i: (i, 0))],
        out_specs=pl.BlockSpec((_BLOCK_ROWS, _EMB), lambda i: (i, 0)),
        out_shape=jax.ShapeDtypeStruct((_VOCAB, _EMB), jnp.float32),
    )(W_emb)
    return out
